# baseline (speedup 1.0000x reference)
import functools

import jax
import jax.numpy as jnp
from jax import lax
from jax.experimental import pallas as pl
from jax.experimental.pallas import tpu as pltpu
from jax.experimental.pallas import tpu_sc as plsc

_B = 16384
_F = 32
_BIAS = 0.1
_NW = 32
_BPW = _B // _NW
_CHUNK = 128
_NCHUNK = _BPW // _CHUNK

_DNUMS = lax.GatherDimensionNumbers(
    offset_dims=(), collapsed_slice_dims=(0,), start_index_map=(0,))


def _rot(x, perm):
  return lax.gather(x, perm, _DNUMS, slice_sizes=(1,),
                    mode=lax.GatherScatterMode.PROMISE_IN_BOUNDS)


def _mf_body(users_hbm, items_hbm, uemb_hbm, iemb_hbm, ub_hbm, ib_hbm,
             out_hbm, uidx_v, iidx_v, urows_v, irows_v, ubias_v, ibias_v,
             out_v, sem):
  wid = lax.axis_index("s") * 2 + lax.axis_index("c")
  base = wid * _BPW

  pltpu.sync_copy(users_hbm.at[pl.ds(base, _BPW)], uidx_v)
  pltpu.sync_copy(items_hbm.at[pl.ds(base, _BPW)], iidx_v)

  copies = []
  for j in range(_NCHUNK):
    s = pl.ds(j * _CHUNK, _CHUNK)
    copies.append(pltpu.async_copy(uemb_hbm.at[uidx_v.at[s]], urows_v.at[s], sem))
    copies.append(pltpu.async_copy(iemb_hbm.at[iidx_v.at[s]], irows_v.at[s], sem))
    copies.append(pltpu.async_copy(ub_hbm.at[uidx_v.at[s]], ubias_v.at[s], sem))
    copies.append(pltpu.async_copy(ib_hbm.at[iidx_v.at[s]], ibias_v.at[s], sem))
  for c in copies:
    c.wait()

  iota16 = lax.iota(jnp.int32, 16)
  perms = [((iota16 + k) & 15)[:, None] for k in (8, 4, 2, 1)]
  lane_masks = [iota16 == k for k in range(16)]

  def group(g, carry):
    rb = g * 16
    acc = ubias_v[pl.ds(rb, 16)] + ibias_v[pl.ds(rb, 16)] + jnp.float32(_BIAS)
    for k in range(16):
      r = rb + k
      p = (urows_v[r, pl.ds(0, 16)] * irows_v[r, pl.ds(0, 16)]
           + urows_v[r, pl.ds(16, 16)] * irows_v[r, pl.ds(16, 16)])
      for pm in perms:
        p = p + _rot(p, pm)
      acc = jnp.where(lane_masks[k], acc + p, acc)
    out_v[pl.ds(rb, 16)] = acc
    return carry

  lax.fori_loop(0, _BPW // 16, group, 0)
  pltpu.sync_copy(out_v, out_hbm.at[pl.ds(base, _BPW)])


@jax.jit
def _mf(users, items, user_emb, item_emb, user_bias_emb, item_bias_emb):
  mesh = plsc.VectorSubcoreMesh(core_axis_name="c", subcore_axis_name="s")
  return pl.kernel(
      _mf_body,
      out_type=jax.ShapeDtypeStruct((_B,), jnp.float32),
      mesh=mesh,
      compiler_params=pltpu.CompilerParams(use_tc_tiling_on_sc=False),
      scratch_types=[
          pltpu.VMEM((_BPW,), jnp.int32),
          pltpu.VMEM((_BPW,), jnp.int32),
          pltpu.VMEM((_BPW, _F), jnp.float32),
          pltpu.VMEM((_BPW, _F), jnp.float32),
          pltpu.VMEM((_BPW,), jnp.float32),
          pltpu.VMEM((_BPW,), jnp.float32),
          pltpu.VMEM((_BPW,), jnp.float32),
          pltpu.SemaphoreType.DMA,
      ],
  )(users, items, user_emb, item_emb,
    user_bias_emb.reshape(-1), item_bias_emb.reshape(-1))


def kernel(users, items, user_emb, item_emb, user_bias_emb, item_bias_emb):
  return _mf(users, items, user_emb, item_emb, user_bias_emb,
             item_bias_emb)
